# Initial kernel scaffold; baseline (speedup 1.0000x reference)
import jax
import jax.numpy as jnp
from jax.experimental import pallas as pl


def kernel(x, in_proj_w, in_proj_b, out_w, out_b, ln1_w, ln1_b, ln2_w, ln2_b, W_up, W_down, router_w):
    raise NotImplementedError("write your pallas kernel here")


# speedup vs baseline: 1.4121x; 1.4121x over previous
import math

import jax
import jax.numpy as jnp
import numpy as np
from jax.experimental import pallas as pl
from jax.experimental.pallas import tpu as pltpu

B, S, D = 1, 2048, 768
H = 12
DH = D // H
DFF = 2048
E = 8
PHI = (1.0 + math.sqrt(5.0)) / 2.0

_r = np.array(
    [[1.0, -1.0, 0.0, 0.0],
     [0.0, 1.0, -1.0, 0.0],
     [0.0, 0.0, 1.0, 0.0],
     [-0.5, 0.5, 0.5, PHI / 2.0]], dtype=np.float32)
_r = _r / np.linalg.norm(_r, axis=1, keepdims=True)
_ROOTS = jnp.asarray(_r)

_pairs = []
for _i in range(16):
    _e1 = _i % E
    _e2 = (_i // 2 + 1) % E
    if _e1 == _e2:
        _e2 = (_e2 + 1) % E
    _pairs.append([_e1, _e2])
_C2E = jnp.asarray(np.array(_pairs, dtype=np.int32))


def _layernorm(x, w, b):
    m = jnp.mean(x, axis=-1, keepdims=True)
    v = jnp.mean((x - m) ** 2, axis=-1, keepdims=True)
    return (x - m) / jnp.sqrt(v + 1e-5) * w + b


def _attention(x, in_proj_w, in_proj_b, out_w, out_b):
    qkv = x @ in_proj_w.T + in_proj_b
    q, k, v = jnp.split(qkv, 3, axis=-1)

    def sh(t):
        return t.reshape(B, S, H, DH).transpose(0, 2, 1, 3)

    q, k, v = sh(q), sh(k), sh(v)
    scores = (q @ k.transpose(0, 1, 3, 2)) / math.sqrt(DH)
    a = jax.nn.softmax(scores, axis=-1)
    o = a @ v
    o = o.transpose(0, 2, 1, 3).reshape(B, S, D)
    return o @ out_w.T + out_b


def _alpha_kernel(wu_ref, wd_ref, au_ref, ad_ref):
    au_ref[...] = jnp.mean(jnp.abs(wu_ref[0])).reshape(1, 1, 1)
    ad_ref[...] = jnp.mean(jnp.abs(wd_ref[0])).reshape(1, 1, 1)


def _erf(x):
    a1, a2, a3, a4, a5 = (0.254829592, -0.284496736, 1.421413741,
                          -1.453152027, 1.061405429)
    p = 0.3275911
    ax = jnp.abs(x)
    t = 1.0 / (1.0 + p * ax)
    poly = t * (a1 + t * (a2 + t * (a3 + t * (a4 + t * a5))))
    y = 1.0 - poly * jnp.exp(-ax * ax)
    return jnp.sign(x) * y


def _gelu(x):
    return 0.5 * x * (1.0 + _erf(x * (1.0 / math.sqrt(2.0))))


TS_MOE = 256


def _moe_kernel(xn_ref, wu_ref, wd_ref, au_ref, ad_ref,
                e0_ref, e1_ref, w1_ref, x2_ref, o_ref, wu_s, wd_s):
    e = pl.program_id(0)
    s = pl.program_id(1)

    @pl.when(s == 0)
    def _tern():
        au = au_ref[0, 0, 0]
        ad = ad_ref[0, 0, 0]

        def body(i, carry):
            ru = i * (DFF // 8)
            wu = wu_ref[0, pl.ds(ru, DFF // 8), :]
            wu_s[pl.ds(ru, DFF // 8), :] = jnp.where(
                jnp.abs(wu) > 0.5 * au, jnp.sign(wu), 0.0).astype(jnp.bfloat16)
            rd = i * (D // 8)
            wd = wd_ref[0, pl.ds(rd, D // 8), :]
            wd_s[pl.ds(rd, D // 8), :] = jnp.where(
                jnp.abs(wd) > 0.5 * ad, jnp.sign(wd), 0.0).astype(jnp.bfloat16)
            return carry

        jax.lax.fori_loop(0, 8, body, 0)

    xn = xn_ref[...]
    rms = jnp.sqrt(jnp.mean(xn * xn, axis=-1, keepdims=True))
    xs = (xn / (rms + 1e-8)).astype(jnp.bfloat16)
    u = jax.lax.dot_general(xs, wu_s[...], (((1,), (1,)), ((), ())),
                            preferred_element_type=jnp.float32)
    u = _gelu(u)
    urms = jnp.sqrt(jnp.mean(u * u, axis=-1, keepdims=True))
    u = (u / (urms + 1e-8)).astype(jnp.bfloat16)
    u = jax.lax.dot_general(u, wd_s[...], (((1,), (1,)), ((), ())),
                            preferred_element_type=jnp.float32)
    w1 = w1_ref[...]
    we = (jnp.where(e0_ref[...] == e, w1, 0.0)
          + jnp.where(e1_ref[...] == e, 1.0 - w1, 0.0))
    contrib = u * we
    row0 = s * TS_MOE

    @pl.when(e == 0)
    def _init():
        o_ref[pl.ds(row0, TS_MOE), :] = x2_ref[pl.ds(row0, TS_MOE), :] + contrib

    @pl.when(e != 0)
    def _acc():
        o_ref[pl.ds(row0, TS_MOE), :] = o_ref[pl.ds(row0, TS_MOE), :] + contrib


def kernel(x, in_proj_w, in_proj_b, out_w, out_b, ln1_w, ln1_b,
           ln2_w, ln2_b, W_up, W_down, router_w):
    residual = x
    h = _layernorm(x, ln1_w, ln1_b)
    h = _attention(h, in_proj_w, in_proj_b, out_w, out_b)
    x2 = h + residual
    x_norm = _layernorm(x2, ln2_w, ln2_b)
    h4 = x_norm @ router_w.T
    h4 = h4 / jnp.maximum(jnp.linalg.norm(h4, axis=-1, keepdims=True), 1e-12)
    dots = h4 @ _ROOTS.T
    bits = (dots >= 0).astype(jnp.int32)
    chamber = (bits[..., 0] + 2 * bits[..., 1]
               + 4 * bits[..., 2] + 8 * bits[..., 3])
    expert_indices = _C2E[chamber]
    confidence = jnp.min(jnp.abs(dots), axis=-1)
    w1 = 0.5 + 0.3 * jax.nn.sigmoid(confidence)
    e0 = expert_indices[..., 0].reshape(S, 1)
    e1 = expert_indices[..., 1].reshape(S, 1)
    w1 = w1.reshape(S, 1)

    alpha_up, alpha_dn = pl.pallas_call(
        _alpha_kernel,
        grid=(E,),
        in_specs=[
            pl.BlockSpec((1, DFF, D), lambda e: (e, 0, 0)),
            pl.BlockSpec((1, D, DFF), lambda e: (e, 0, 0)),
        ],
        out_specs=[
            pl.BlockSpec((1, 1, 1), lambda e: (e, 0, 0)),
            pl.BlockSpec((1, 1, 1), lambda e: (e, 0, 0)),
        ],
        out_shape=[
            jax.ShapeDtypeStruct((E, 1, 1), jnp.float32),
            jax.ShapeDtypeStruct((E, 1, 1), jnp.float32),
        ],
    )(W_up, W_down)

    out = pl.pallas_call(
        _moe_kernel,
        grid=(E, S // TS_MOE),
        in_specs=[
            pl.BlockSpec((TS_MOE, D), lambda e, s: (s, 0)),
            pl.BlockSpec((1, DFF, D), lambda e, s: (e, 0, 0)),
            pl.BlockSpec((1, D, DFF), lambda e, s: (e, 0, 0)),
            pl.BlockSpec((1, 1, 1), lambda e, s: (e, 0, 0)),
            pl.BlockSpec((1, 1, 1), lambda e, s: (e, 0, 0)),
            pl.BlockSpec((TS_MOE, 1), lambda e, s: (s, 0)),
            pl.BlockSpec((TS_MOE, 1), lambda e, s: (s, 0)),
            pl.BlockSpec((TS_MOE, 1), lambda e, s: (s, 0)),
            pl.BlockSpec((S, D), lambda e, s: (0, 0)),
        ],
        out_specs=pl.BlockSpec((S, D), lambda e, s: (0, 0)),
        out_shape=jax.ShapeDtypeStruct((S, D), jnp.float32),
        scratch_shapes=[
            pltpu.VMEM((DFF, D), jnp.bfloat16),
            pltpu.VMEM((D, DFF), jnp.bfloat16),
        ],
    )(x_norm.reshape(S, D), W_up, W_down, alpha_up, alpha_dn,
      e0, e1, w1, x2.reshape(S, D))

    return out.reshape(B, S, D)
